# baseline (speedup 1.0000x reference)
import functools

import jax
import jax.numpy as jnp
from jax import lax
from jax.experimental import pallas as pl
from jax.experimental.pallas import tpu as pltpu
from jax.experimental.pallas import tpu_sc as plsc

N, W = 131072, 64
N2, W2 = N // 2, 2 * W
NC, NS = 2, 16
NWORK = NC * NS
RPW = N // NWORK
CH2 = 128
MCH = 2 * CH2
NCHUNK = RPW // MCH


def _perm(vec, idx):
    dn = lax.GatherDimensionNumbers(
        offset_dims=(), collapsed_slice_dims=(0,), start_index_map=(0,))
    return lax.gather(vec, idx[:, None], dn, slice_sizes=(1,),
                      mode=lax.GatherScatterMode.PROMISE_IN_BOUNDS)


def _make_sc_kernel():
    mesh = plsc.VectorSubcoreMesh(core_axis_name="c", subcore_axis_name="s")

    @functools.partial(
        pl.kernel,
        mesh=mesh,
        compiler_params=pltpu.CompilerParams(
            needs_layout_passes=False, skip_device_barrier=True),
        out_type=[
            jax.ShapeDtypeStruct((N,), jnp.float32),
            jax.ShapeDtypeStruct((N,), jnp.float32),
        ],
        scratch_types=[
            pltpu.VMEM((CH2, W2), jnp.float32),
            pltpu.VMEM((CH2, W2), jnp.float32),
            pltpu.VMEM((RPW,), jnp.float32),
            pltpu.VMEM((RPW,), jnp.float32),
            pltpu.VMEM((W,), jnp.float32),
            pltpu.SemaphoreType.DMA,
            pltpu.SemaphoreType.DMA,
        ],
    )
    def sc_kernel(key_hbm, mem_hbm, dot_hbm, sq_hbm,
                  buf0, buf1, dot_loc, sq_loc, kbuf, sem0, sem1):
        wid = lax.axis_index("s") * NC + lax.axis_index("c")
        base2 = wid * (N2 // NWORK)

        pltpu.sync_copy(key_hbm, kbuf)
        kv = [kbuf[pl.ds(16 * j, 16)] for j in range(4)]

        lanes = lax.iota(jnp.int32, 16)
        bufs = (buf0, buf1)
        sems = (sem0, sem1)
        half = NCHUNK // 2

        pltpu.async_copy(mem_hbm.at[pl.ds(base2, CH2)], buf0, sem0)
        pltpu.async_copy(mem_hbm.at[pl.ds(base2 + CH2, CH2)], buf1, sem1)

        def process_chunk(ch, buf):
            def group_body(g, carry):
                rows = g * 16 + lanes
                rows2 = lax.shift_right_logical(rows, 1)
                rowpar = lax.shift_left(rows & 1, 6)
                rlane = rows & 15
                zero = jnp.zeros((16,), jnp.float32)
                dots = [zero] * 8
                sqs = [zero] * 8
                for j in range(4):
                    for t in range(16):
                        a = 2 * j + (t & 1)
                        colp = (rlane + t) & 15
                        v = plsc.load_gather(
                            buf, [rows2, rowpar + colp + 16 * j])
                        kc = _perm(kv[j], colp)
                        dots[a] = dots[a] + v * kc
                        sqs[a] = sqs[a] + v * v
                dot = (((dots[0] + dots[1]) + (dots[2] + dots[3]))
                       + ((dots[4] + dots[5]) + (dots[6] + dots[7])))
                sq = (((sqs[0] + sqs[1]) + (sqs[2] + sqs[3]))
                      + ((sqs[4] + sqs[5]) + (sqs[6] + sqs[7])))
                off = ch * MCH + g * 16
                dot_loc[pl.ds(off, 16)] = dot
                sq_loc[pl.ds(off, 16)] = sq
                return carry

            lax.fori_loop(0, MCH // 16, group_body, 0)

        def pair_body(i, carry):
            for b in range(2):
                ch = 2 * i + b
                pltpu.make_async_copy(
                    mem_hbm.at[pl.ds(base2, CH2)], bufs[b], sems[b]).wait()
                process_chunk(ch, bufs[b])

                @pl.when(i + 1 < half)
                def _():
                    pltpu.async_copy(
                        mem_hbm.at[pl.ds(base2 + (ch + 2) * CH2, CH2)],
                        bufs[b], sems[b])

            return carry

        lax.fori_loop(0, half, pair_body, 0)

        rbase = wid * RPW
        pltpu.sync_copy(dot_loc, dot_hbm.at[pl.ds(rbase, RPW)])
        pltpu.sync_copy(sq_loc, sq_hbm.at[pl.ds(rbase, RPW)])

    return sc_kernel


_sc_kernel = _make_sc_kernel()

BR = 256
NBB = N // 128 // BR


def _tc_body(key_ref, s_ref, dot_ref, sq_ref, out_ref, acc_ref):
    i = pl.program_id(0)
    kv = key_ref[...]
    n1 = jnp.sqrt(jnp.sum(kv * kv))
    d = dot_ref[...]
    q = sq_ref[...]
    denom = jnp.maximum(n1 * jnp.sqrt(q), 1e-8)
    e = jnp.exp(d / denom * s_ref[0])
    bsum = jnp.sum(e)

    @pl.when(i == 0)
    def _():
        acc_ref[0] = bsum

    @pl.when(i > 0)
    def _():
        acc_ref[0] = acc_ref[0] + bsum

    out_ref[pl.ds(i * BR, BR), :] = e

    @pl.when(i == NBB - 1)
    def _():
        out_ref[...] = out_ref[...] * (1.0 / acc_ref[0])


def _tc_finish(write_key, write_strength, dot2d, sq2d):
    return pl.pallas_call(
        _tc_body,
        grid=(NBB,),
        in_specs=[
            pl.BlockSpec((1, W), lambda i: (0, 0)),
            pl.BlockSpec(memory_space=pltpu.SMEM),
            pl.BlockSpec((BR, 128), lambda i: (i, 0)),
            pl.BlockSpec((BR, 128), lambda i: (i, 0)),
        ],
        out_specs=pl.BlockSpec((N // 128, 128), lambda i: (0, 0)),
        out_shape=jax.ShapeDtypeStruct((N // 128, 128), jnp.float32),
        scratch_shapes=[pltpu.SMEM((1,), jnp.float32)],
    )(write_key, write_strength, dot2d, sq2d)


def kernel(write_key, write_strength, memory):
    mem2 = memory.reshape(N2, W2)
    dot, sq = _sc_kernel(write_key.reshape(W), mem2)
    out = _tc_finish(write_key, write_strength,
                     dot.reshape(N // 128, 128), sq.reshape(N // 128, 128))
    return out.reshape(N)


# speedup vs baseline: 1.4453x; 1.4453x over previous
import jax
import jax.numpy as jnp
from jax.experimental import pallas as pl
from jax.experimental.pallas import tpu as pltpu

N, W = 131072, 64
BLK = 4096
NB = N // BLK
OR = N // 128
BRO = BLK // 128


def _body(key_ref, s_ref, mem_ref, out_ref, dacc, qacc, acc_ref):
    i = pl.program_id(0)
    mb = mem_ref[...]
    kv = key_ref[...]
    dot = jnp.sum(mb * kv, axis=1)
    sq = jnp.sum(mb * mb, axis=1)
    dacc[pl.ds(i * BRO, BRO), :] = dot.reshape(BRO, 128)
    qacc[pl.ds(i * BRO, BRO), :] = sq.reshape(BRO, 128)

    @pl.when(i == NB - 1)
    def _():
        n1 = jnp.sqrt(jnp.sum(kv * kv))
        d = dacc[...]
        q = qacc[...]
        denom = jnp.maximum(n1 * jnp.sqrt(q), 1e-8)
        e = jnp.exp(d / denom * s_ref[0])
        out_ref[...] = e * (1.0 / jnp.sum(e))


def kernel(write_key, write_strength, memory):
    out = pl.pallas_call(
        _body,
        grid=(NB,),
        in_specs=[
            pl.BlockSpec((1, W), lambda i: (0, 0)),
            pl.BlockSpec(memory_space=pltpu.SMEM),
            pl.BlockSpec((BLK, W), lambda i: (i, 0)),
        ],
        out_specs=pl.BlockSpec((OR, 128), lambda i: (0, 0)),
        out_shape=jax.ShapeDtypeStruct((OR, 128), jnp.float32),
        scratch_shapes=[
            pltpu.VMEM((OR, 128), jnp.float32),
            pltpu.VMEM((OR, 128), jnp.float32),
            pltpu.SMEM((1,), jnp.float32),
        ],
    )(write_key, write_strength, memory)
    return out.reshape(N)


# speedup vs baseline: 1.5244x; 1.0548x over previous
import jax
import jax.numpy as jnp
from jax.experimental import pallas as pl
from jax.experimental.pallas import tpu as pltpu

N, W = 131072, 64
BLK = 4096
NB = N // BLK
OR = N // 128
BRO = BLK // 128


def _body(key_ref, s_ref, mem_ref, out_ref, dacc, qacc, acc_ref):
    i = pl.program_id(0)
    mb = mem_ref[...]
    kv = key_ref[...]
    dot = jnp.sum(mb * kv, axis=2)
    sq = jnp.sum(mb * mb, axis=2)
    dacc[pl.ds(i * BRO, BRO), :] = dot
    qacc[pl.ds(i * BRO, BRO), :] = sq

    @pl.when(i == NB - 1)
    def _():
        kf = kv.reshape(1, W)
        n1 = jnp.sqrt(jnp.sum(kf * kf))
        d = dacc[...]
        q = qacc[...]
        denom = jnp.maximum(n1 * jnp.sqrt(q), 1e-8)
        e = jnp.exp(d / denom * s_ref[0])
        out_ref[...] = e * (1.0 / jnp.sum(e))


def kernel(write_key, write_strength, memory):
    mem3 = memory.reshape(OR, 128, W)
    key3 = write_key.reshape(1, 1, W)
    out = pl.pallas_call(
        _body,
        grid=(NB,),
        in_specs=[
            pl.BlockSpec((1, 1, W), lambda i: (0, 0, 0)),
            pl.BlockSpec(memory_space=pltpu.SMEM),
            pl.BlockSpec((BRO, 128, W), lambda i: (i, 0, 0)),
        ],
        out_specs=pl.BlockSpec((OR, 128), lambda i: (0, 0)),
        out_shape=jax.ShapeDtypeStruct((OR, 128), jnp.float32),
        scratch_shapes=[
            pltpu.VMEM((OR, 128), jnp.float32),
            pltpu.VMEM((OR, 128), jnp.float32),
            pltpu.SMEM((1,), jnp.float32),
        ],
    )(key3, write_strength, mem3)
    return out.reshape(N)
